# baseline (speedup 1.0000x reference)
import functools

import jax
import jax.numpy as jnp
from jax import lax
from jax.experimental import pallas as pl
from jax.experimental.pallas import tpu as pltpu
from jax.experimental.pallas import tpu_sc as plsc

VOCAB = 1000000
EMBED = 32
B = 16384
MAX_SUBWORDS = 10

NC = 2
NS = 16
NW = NC * NS
BPW = B // NW
C = 64
NCHUNK = BPW // C
G = MAX_SUBWORDS
PACK = 128 // EMBED
ZROW = C * G


def _body(table_hbm, ids_hbm, lens_hbm, out_hbm, ids_v, idx_v, rows_v,
          lens_v, out_v, sem):
    wid = lax.axis_index("s") * NC + lax.axis_index("c")
    riota = jax.lax.iota(jnp.int32, 16)
    zero16 = jnp.zeros((16,), jnp.float32)
    for h in range(128 // 16):
        rows_v[ZROW, pl.ds(h * 16, 16)] = zero16

    def chunk_body(chunk, _):
        wbase = wid * BPW + chunk * C
        pltpu.sync_copy(ids_hbm.at[wid * NCHUNK + chunk], ids_v)
        pltpu.sync_copy(lens_hbm.at[pl.ds(wbase, C)], lens_v)
        for g in range(G):
            for k in range(C // 16):
                idx_v[g, pl.ds(k * 16, 16)] = (
                    ids_v[g, pl.ds(k * 16, 16)] >> 2)
        copies = [
            pltpu.async_copy(table_hbm.at[idx_v.at[g]],
                             rows_v.at[pl.ds(g * C, C)], sem)
            for g in range(G)
        ]
        for cp in copies:
            cp.wait()

        def group_body(k, _):
            kb = k * 16
            lens16 = lens_v[pl.ds(kb, 16)]
            linv16 = 1.0 / lens16.astype(jnp.float32)
            acc = [zero16] * EMBED
            for j in range(G):
                ids16 = ids_v[j, pl.ds(kb, 16)]
                off16 = (ids16 & (PACK - 1)) * EMBED
                rowj = jnp.where(lens16 > j, j * C + kb + riota, ZROW)
                for e in range(EMBED):
                    v = plsc.load_gather(rows_v, [rowj, off16 + e])
                    acc[e] = acc[e] + v
            widx = kb + riota
            for e in range(EMBED):
                plsc.store_scatter(out_v, [widx, jnp.full((16,), e, jnp.int32)],
                                   acc[e] * linv16)
            return 0

        lax.fori_loop(0, C // 16, group_body, 0)
        pltpu.sync_copy(out_v, out_hbm.at[pl.ds(wbase, C)])
        return 0

    lax.fori_loop(0, NCHUNK, chunk_body, 0)


@functools.partial(jax.jit, static_argnames=())
def kernel(subword_ids, lengths, table):
    table4 = table.reshape(VOCAB // PACK, EMBED * PACK)
    ids3d = subword_ids.reshape(NW * NCHUNK, C, G).transpose(0, 2, 1)
    mesh = plsc.VectorSubcoreMesh(core_axis_name="c", subcore_axis_name="s")
    fn = pl.kernel(
        _body,
        mesh=mesh,
        out_type=jax.ShapeDtypeStruct((B, EMBED), jnp.float32),
        scratch_types=[
            pltpu.VMEM((G, C), jnp.int32),
            pltpu.VMEM((G, C), jnp.int32),
            pltpu.VMEM((C * G + 8, EMBED * PACK), jnp.float32),
            pltpu.VMEM((C,), jnp.int32),
            pltpu.VMEM((C, EMBED), jnp.float32),
            pltpu.SemaphoreType.DMA,
        ],
        compiler_params=pltpu.CompilerParams(needs_layout_passes=False),
    )
    return fn(table4, ids3d, lengths)
